# baseline (speedup 1.0000x reference)
import jax
import jax.numpy as jnp
from jax.experimental import pallas as pl
from jax.experimental.pallas import tpu as pltpu
from functools import partial


def _crps_body(f_ref, obs_ref, w_ref, out_ref, R_ref, u_ref, *, S, B):
    t = pl.program_id(0)
    nsteps = pl.num_programs(0)

    f_full = f_ref[0, :]
    obs = obs_ref[0, :]
    w_blk = w_ref[0]

    ones = jnp.ones((S,), jnp.float32)

    R_blk = jax.lax.dot_general(
        w_blk, ones, (((1,), (0,)), ((), ())),
        preferred_element_type=jnp.float32)
    R_ref[0, pl.ds(t * B, B)] = R_blk

    j_glob = t * B + jax.lax.broadcasted_iota(jnp.int32, (B, S), 0)
    k_glob = jax.lax.broadcasted_iota(jnp.int32, (B, S), 1)
    f_j = f_ref[0, pl.ds(t * B, B)]
    fj_b = f_j[:, None]
    fk_b = f_full[None, :]
    C = jnp.where(
        (fk_b < fj_b) | ((fk_b == fj_b) & (k_glob <= j_glob)),
        1.0, 0.0).astype(jnp.float32)

    s_j = jax.lax.dot_general(C, f_full, (((1,), (0,)), ((), ())),
                              preferred_element_type=jnp.float32)
    r_j = jax.lax.dot_general(C, ones, (((1,), (0,)), ((), ())),
                              preferred_element_type=jnp.float32) - 1.0

    i_glob = jax.lax.broadcasted_iota(jnp.int32, (B, S), 1)
    r_i32 = r_j.astype(jnp.int32)
    M = jnp.where(r_i32[:, None] == i_glob, 1.0, 0.0)
    obs_g = jax.lax.dot_general(M, obs, (((1,), (0,)), ((), ())),
                                preferred_element_type=jnp.float32)
    ind = (f_j > obs_g).astype(jnp.float32)
    t_j = (s_j / S - ind) ** 2

    u_inc = jax.lax.dot_general(t_j, M, (((0,), (0,)), ((), ())),
                                preferred_element_type=jnp.float32)

    @pl.when(t == 0)
    def _():
        u_ref[0, :] = u_inc

    @pl.when(t > 0)
    def _():
        u_ref[0, :] += u_inc

    @pl.when(t == nsteps - 1)
    def _():
        crps = jnp.sum(R_ref[0, :] * u_ref[0, :]) / (S * S)
        out_ref[...] = jnp.reshape(crps, (1, 1))


def _crps_pallas(forecast, observations, weights, *, B, interpret=False):
    S = forecast.shape[-1]
    nsteps = S // B
    obs2d = observations.reshape(1, S)
    out = pl.pallas_call(
        partial(_crps_body, S=S, B=B),
        grid=(nsteps,),
        in_specs=[
            pl.BlockSpec((1, S), lambda t: (0, 0)),
            pl.BlockSpec((1, S), lambda t: (0, 0)),
            pl.BlockSpec((1, B, S), lambda t: (0, t, 0)),
        ],
        out_specs=pl.BlockSpec((1, 1), lambda t: (0, 0)),
        out_shape=jax.ShapeDtypeStruct((1, 1), jnp.float32),
        scratch_shapes=[
            pltpu.VMEM((1, S), jnp.float32),
            pltpu.VMEM((1, S), jnp.float32),
        ],
        interpret=interpret,
    )(forecast, obs2d, weights)
    return out[0, 0]


def kernel(forecast, observations, weights):
    return _crps_pallas(forecast, observations, weights, B=256)


# speedup vs baseline: 2.0840x; 2.0840x over previous
import jax
import jax.numpy as jnp
from jax.experimental import pallas as pl
from jax.experimental.pallas import tpu as pltpu
from functools import partial

_C = 128


def _xchg_axis0(A, m, k, ig, ig0):
    n0 = A.shape[0]
    up = jnp.concatenate([A[m:], A[:m]], axis=0)
    dn = jnp.concatenate([A[n0 - m:], A[:n0 - m]], axis=0)
    pbit = (ig0 & m) == 0
    P = jnp.where(pbit, up, dn)
    mn = jnp.minimum(A, P)
    mx = jnp.maximum(A, P)
    dirn = (ig & k) == 0
    take_min = dirn == pbit
    return jnp.where(take_min, mn, mx)


def _crps_body(f_ref, obs_ref, w_ref, out_ref, y_ref, d_ref, R_ref, *, S, B):
    t = pl.program_id(0)
    nsteps = pl.num_programs(0)
    _R = S // _C
    NP = S.bit_length() - 1

    w_blk = w_ref[0]
    ones = jnp.ones((S,), jnp.float32)
    R_blk = jax.lax.dot_general(
        w_blk, ones, (((1,), (0,)), ((), ())),
        preferred_element_type=jnp.float32)
    rows = B // _C
    R_ref[pl.ds(t * rows, rows), :] = R_blk.reshape(rows, _C)

    ig_x = (jax.lax.broadcasted_iota(jnp.int32, (_R, _C), 0) * _C
            + jax.lax.broadcasted_iota(jnp.int32, (_R, _C), 1))
    ig_y = (jax.lax.broadcasted_iota(jnp.int32, (_C, _R), 0)
            + jax.lax.broadcasted_iota(jnp.int32, (_C, _R), 1) * _C)
    ig0_x = jax.lax.broadcasted_iota(jnp.int32, (_R, _C), 0)
    ig0_y = jax.lax.broadcasted_iota(jnp.int32, (_C, _R), 0)
    @pl.when(t == 0)
    def _():
        y_ref[...] = f_ref[...].T

    for p in range(NP):
        k = 1 << (p + 1)

        @pl.when(t == p)
        def _(k=k):
            jj = k // 2
            if jj >= _C:
                x = y_ref[...].T
                while jj >= _C:
                    x = _xchg_axis0(x, jj // _C, k, ig_x, ig0_x)
                    jj //= 2
                y_ref[...] = x.T
            y = y_ref[...]
            while jj >= 1:
                y = _xchg_axis0(y, jj, k, ig_y, ig0_y)
                jj //= 2
            y_ref[...] = y

    @pl.when(t == NP)
    def _():
        sf = y_ref[...].T
        a_le_b = (jax.lax.broadcasted_iota(jnp.int32, (_C, _C), 0)
                  <= jax.lax.broadcasted_iota(jnp.int32, (_C, _C), 1))
        L = jnp.where(a_le_b, 1.0, 0.0)
        cs_in = jax.lax.dot_general(
            sf, L, (((1,), (0,)), ((), ())),
            preferred_element_type=jnp.float32)
        rowsum = jax.lax.dot_general(
            sf, jnp.ones((_C,), jnp.float32), (((1,), (0,)), ((), ())),
            preferred_element_type=jnp.float32)
        a_lt_b = (jax.lax.broadcasted_iota(jnp.int32, (_R, _R), 0)
                  < jax.lax.broadcasted_iota(jnp.int32, (_R, _R), 1))
        U = jnp.where(a_lt_b, 1.0, 0.0)
        rp = jax.lax.dot_general(
            rowsum, U, (((0,), (0,)), ((), ())),
            preferred_element_type=jnp.float32)
        cs = cs_in + rp[:, None]
        ind = (sf > obs_ref[...]).astype(jnp.float32)
        d_ref[...] = (cs * (1.0 / S) - ind) ** 2

    @pl.when(t == nsteps - 1)
    def _():
        crps = jnp.sum(R_ref[...] * d_ref[...]) / (S * S)
        out_ref[...] = jnp.reshape(crps, (1, 1))


def _crps_pallas(forecast, observations, weights, *, B, interpret=False):
    S = forecast.size
    _R = S // _C
    nsteps = S // B
    assert nsteps > S.bit_length() - 1
    f2d = forecast.reshape(_R, _C)
    obs2d = observations.reshape(_R, _C)
    out = pl.pallas_call(
        partial(_crps_body, S=S, B=B),
        grid=(nsteps,),
        in_specs=[
            pl.BlockSpec((_R, _C), lambda t: (0, 0)),
            pl.BlockSpec((_R, _C), lambda t: (0, 0)),
            pl.BlockSpec((1, B, S), lambda t: (0, t, 0)),
        ],
        out_specs=pl.BlockSpec((1, 1), lambda t: (0, 0)),
        out_shape=jax.ShapeDtypeStruct((1, 1), jnp.float32),
        scratch_shapes=[
            pltpu.VMEM((_C, _R), jnp.float32),
            pltpu.VMEM((_R, _C), jnp.float32),
            pltpu.VMEM((_R, _C), jnp.float32),
        ],
        interpret=interpret,
    )(f2d, obs2d, weights)
    return out[0, 0]


def kernel(forecast, observations, weights):
    return _crps_pallas(forecast, observations, weights, B=256)
